# baseline (speedup 1.0000x reference)
import functools

import jax
import jax.numpy as jnp
from jax import lax
from jax.experimental import pallas as pl
from jax.experimental.pallas import tpu as pltpu
from jax.experimental.pallas import tpu_sc as plsc

B = 8
N = 2048
K = 16
A = 4
CH = 28
L = 16
NWORKERS = 32
QPT = (B * N) // NWORKERS
QB = 16

_mesh = plsc.VectorSubcoreMesh(core_axis_name="c", subcore_axis_name="s")


def _worker_id():
    return lax.axis_index("s") * 2 + lax.axis_index("c")


def _splat(x, dtype=jnp.int32):
    return jnp.broadcast_to(jnp.asarray(x, dtype), (L,))


def _bf16_round(v):
    b = plsc.bitcast(v, jnp.int32)
    lsb = (b >> 16) & 1
    r = (b + 0x7FFF + lsb) & jnp.int32(-65536)
    return plsc.bitcast(r, jnp.float32)


@functools.partial(
    pl.kernel,
    mesh=_mesh,
    compiler_params=pltpu.CompilerParams(needs_layout_passes=False, use_tc_tiling_on_sc=False),
    out_type=[
        jax.ShapeDtypeStruct((B, N, K), jnp.int32),
        jax.ShapeDtypeStruct((B, N, K), jnp.float32),
    ],
    scratch_types=[
        pltpu.VMEM((N, 3), jnp.float32),
        pltpu.VMEM((N,), jnp.float32),
        pltpu.VMEM((N,), jnp.float32),
        pltpu.VMEM((N,), jnp.float32),
        pltpu.VMEM((N,), jnp.float32),
        pltpu.VMEM((QPT, K), jnp.int32),
        pltpu.VMEM((QPT, K), jnp.float32),
    ],
)
def _topk_kernel(xyz_hbm, idx_hbm, d2_hbm,
                 xyzv, x2v, xbv, ybv, zbv, oidx, od2):
    wid = _worker_id()
    b = wid // (N // QPT)
    base = (wid % (N // QPT)) * QPT
    pltpu.sync_copy(xyz_hbm.at[b], xyzv)
    iota = lax.iota(jnp.int32, L)
    inf = jnp.broadcast_to(jnp.float32(jnp.inf), (L,))

    def build(c, carry):
        s = c * L
        m = iota + _splat(s)
        xs = plsc.load_gather(xyzv, [m, _splat(0)])
        ys = plsc.load_gather(xyzv, [m, _splat(1)])
        zs = plsc.load_gather(xyzv, [m, _splat(2)])
        x2v[pl.ds(s, L)] = xs * xs + ys * ys + zs * zs
        xbv[pl.ds(s, L)] = _bf16_round(xs)
        ybv[pl.ds(s, L)] = _bf16_round(ys)
        zbv[pl.ds(s, L)] = _bf16_round(zs)
        return carry

    lax.fori_loop(0, N // L, build, 0)

    GQ = 8

    def per_group(g, carry):
        n0 = base + g * GQ
        qs = []
        for t in range(GQ):
            nsp = _splat(n0 + t)
            qs.append((plsc.load_gather(x2v, [nsp]),
                       plsc.load_gather(xbv, [nsp]),
                       plsc.load_gather(ybv, [nsp]),
                       plsc.load_gather(zbv, [nsp])))

        def chunk(j, st):
            s = j * L
            xb = xbv[pl.ds(s, L)]
            yb = ybv[pl.ds(s, L)]
            zb = zbv[pl.ds(s, L)]
            c2 = x2v[pl.ds(s, L)]
            d2s, hits = [], []
            anyhit = None
            for t in range(GQ):
                q2, qx, qy, qz = qs[t]
                dot = xb * qx + yb * qy + zb * qz
                d2 = jnp.maximum((q2 + c2) - 2.0 * dot, 0.0)
                d2s.append(d2)
                m = d2 < st[t][2]
                hits.append(m)
                anyhit = m if anyhit is None else (anyhit | m)

            def do_merges(st):
                out = []
                for t in range(GQ):
                    def merge_t(stt, t=t):
                        T, TI, _ = stt
                        ck, ci = plsc.sort_key_val(d2s[t], iota + _splat(s))
                        rk = lax.rev(ck, (0,))
                        ri = lax.rev(ci, (0,))
                        m = jnp.where(T <= rk, jnp.int32(-1), jnp.int32(0))
                        tb = plsc.bitcast(T, jnp.int32)
                        rb = plsc.bitcast(rk, jnp.int32)
                        nk = plsc.bitcast((tb & m) | (rb & ~m), jnp.float32)
                        ni = (TI & m) | (ri & ~m)
                        T2, TI2 = plsc.sort_key_val(nk, ni)
                        return T2, TI2, jnp.broadcast_to(jnp.max(T2), (L,))

                    out.append(lax.cond(jnp.any(hits[t]), merge_t,
                                        lambda s_: s_, st[t]))
                return tuple(out)

            return lax.cond(jnp.any(anyhit), do_merges, lambda s_: s_, st)

        init = tuple((inf, iota, inf) for _ in range(GQ))
        st = lax.fori_loop(0, N // L, chunk, init)
        for t in range(GQ):
            T, TI, _ = st[t]
            oidx[g * GQ + t] = TI
            od2[g * GQ + t] = T
        return carry

    lax.fori_loop(0, QPT // GQ, per_group, 0)
    pltpu.sync_copy(oidx, idx_hbm.at[b, pl.ds(base, QPT)])
    pltpu.sync_copy(od2, d2_hbm.at[b, pl.ds(base, QPT)])


@functools.partial(
    pl.kernel,
    mesh=_mesh,
    compiler_params=pltpu.CompilerParams(needs_layout_passes=False, use_tc_tiling_on_sc=False),
    out_type=jax.ShapeDtypeStruct((B, N, K, CH), jnp.float32),
    scratch_types=[
        pltpu.VMEM((N, 3), jnp.float32),
        pltpu.VMEM((N,), jnp.float32),
        pltpu.VMEM((N,), jnp.float32),
        pltpu.VMEM((N,), jnp.float32),
        pltpu.VMEM((N,), jnp.float32),
        pltpu.VMEM((N, K), jnp.int32),
        pltpu.VMEM((N, K), jnp.float32),
        pltpu.VMEM((N, 8), jnp.float32),
        pltpu.VMEM((QB, K, CH), jnp.float32),
    ],
)
def _feat_kernel(xyz_hbm, idx_hbm, d2_hbm, feat_hbm,
                 xyzv, x2v, xbv, ybv, zbv, idxv, d2v, intrav, outv):
    wid = _worker_id()
    b = wid // (N // QPT)
    base = (wid % (N // QPT)) * QPT
    pltpu.sync_copy(xyz_hbm.at[b], xyzv)
    pltpu.sync_copy(idx_hbm.at[b], idxv)
    pltpu.sync_copy(d2_hbm.at[b], d2v)
    iota = lax.iota(jnp.int32, L)

    def build(c, carry):
        s = c * L
        m = iota + _splat(s)
        xs = plsc.load_gather(xyzv, [m, _splat(0)])
        ys = plsc.load_gather(xyzv, [m, _splat(1)])
        zs = plsc.load_gather(xyzv, [m, _splat(2)])
        x2v[pl.ds(s, L)] = xs * xs + ys * ys + zs * zs
        xbv[pl.ds(s, L)] = _bf16_round(xs)
        ybv[pl.ds(s, L)] = _bf16_round(ys)
        zbv[pl.ds(s, L)] = _bf16_round(zs)
        return carry

    lax.fori_loop(0, N // L, build, 0)

    def gxyz(idx):
        return (plsc.load_gather(xbv, [idx]),
                plsc.load_gather(ybv, [idx]),
                plsc.load_gather(zbv, [idx]),
                plsc.load_gather(x2v, [idx]))

    def d2_of(p, q):
        dot = p[0] * q[0] + p[1] * q[1] + p[2] * q[2]
        return jnp.maximum((p[3] + q[3]) - 2.0 * dot, 0.0)

    def intra_chunk(c, carry):
        m = iota + _splat(c * L)
        anc = [plsc.load_gather(idxv, [m, _splat(i)]) for i in (1, 2, 3)]
        p = [gxyz(a) for a in anc]
        for ci in range(3):
            g = plsc.load_gather(d2v, [m, _splat(ci + 1)])
            plsc.store_scatter(intrav, [m, _splat(ci)], g)
        pairs = ((0, 1), (0, 2), (1, 2))
        for ci, (i, j) in enumerate(pairs):
            plsc.store_scatter(intrav, [m, _splat(3 + ci)], d2_of(p[i], p[j]))
        return carry

    lax.fori_loop(0, N // L, intra_chunk, 0)

    def per_block(g, carry):
        def per_query(q, carry2):
            n = base + g * QB + q
            nsp = _splat(n)
            nbr = idxv[n]
            qs = _splat(q)
            own = []
            for i in range(A):
                ai = plsc.load_gather(idxv, [nsp, _splat(i)])
                own.append(gxyz(ai))
            na = []
            for j in range(A):
                bj = plsc.load_gather(idxv, [nbr, _splat(j)])
                na.append(gxyz(bj))
            for ci in range(6):
                v = plsc.load_gather(intrav, [nsp, _splat(ci)])
                plsc.store_scatter(outv, [qs, iota, _splat(ci)], v)
            for ci in range(6):
                v = plsc.load_gather(intrav, [nbr, _splat(ci)])
                plsc.store_scatter(outv, [qs, iota, _splat(6 + ci)], v)
            for i in range(A):
                for j in range(A):
                    v = d2_of(own[i], na[j])
                    plsc.store_scatter(
                        outv, [qs, iota, _splat(12 + i * A + j)], v)
            return carry2

        lax.fori_loop(0, QB, per_query, 0)
        pltpu.sync_copy(outv, feat_hbm.at[b, pl.ds(base + g * QB, QB)])
        return carry

    lax.fori_loop(0, QPT // QB, per_block, 0)


_SQRT_ROWS = (B * N * K * CH) // 128
_SQRT_BLK = 2048


def _sqrt_body(x_ref, o_ref):
    o_ref[...] = jnp.sqrt(x_ref[...])


_sqrt_call = pl.pallas_call(
    _sqrt_body,
    out_shape=jax.ShapeDtypeStruct((_SQRT_ROWS, 128), jnp.float32),
    grid=(_SQRT_ROWS // _SQRT_BLK,),
    in_specs=[pl.BlockSpec((_SQRT_BLK, 128), lambda i: (i, 0))],
    out_specs=pl.BlockSpec((_SQRT_BLK, 128), lambda i: (i, 0)),
)


def kernel(xyz):
    idx, d2 = _topk_kernel(xyz)
    feat_d2 = _feat_kernel(xyz, idx, d2)
    feat = _sqrt_call(feat_d2.reshape(_SQRT_ROWS, 128)).reshape(B, N, K, CH)
    return feat, idx


# speedup vs baseline: 1.0678x; 1.0678x over previous
import functools

import jax
import jax.numpy as jnp
from jax import lax
from jax.experimental import pallas as pl
from jax.experimental.pallas import tpu as pltpu
from jax.experimental.pallas import tpu_sc as plsc

B = 8
N = 2048
K = 16
A = 4
CH = 28
L = 16
NWORKERS = 32
QPT = (B * N) // NWORKERS
QB = 16

_mesh = plsc.VectorSubcoreMesh(core_axis_name="c", subcore_axis_name="s")


def _worker_id():
    return lax.axis_index("s") * 2 + lax.axis_index("c")


def _splat(x, dtype=jnp.int32):
    return jnp.broadcast_to(jnp.asarray(x, dtype), (L,))


def _bf16_round(v):
    b = plsc.bitcast(v, jnp.int32)
    lsb = (b >> 16) & 1
    r = (b + 0x7FFF + lsb) & jnp.int32(-65536)
    return plsc.bitcast(r, jnp.float32)


@functools.partial(
    pl.kernel,
    mesh=_mesh,
    compiler_params=pltpu.CompilerParams(needs_layout_passes=False, use_tc_tiling_on_sc=False),
    out_type=[
        jax.ShapeDtypeStruct((B, N, K), jnp.int32),
        jax.ShapeDtypeStruct((B, N, K), jnp.float32),
    ],
    scratch_types=[
        pltpu.VMEM((N, 3), jnp.float32),
        pltpu.VMEM((N,), jnp.float32),
        pltpu.VMEM((N,), jnp.float32),
        pltpu.VMEM((N,), jnp.float32),
        pltpu.VMEM((N,), jnp.float32),
        pltpu.VMEM((QPT, K), jnp.int32),
        pltpu.VMEM((QPT, K), jnp.float32),
    ],
)
def _topk_kernel(xyz_hbm, idx_hbm, d2_hbm,
                 xyzv, x2v, xbv, ybv, zbv, oidx, od2):
    wid = _worker_id()
    b = wid // (N // QPT)
    base = (wid % (N // QPT)) * QPT
    pltpu.sync_copy(xyz_hbm.at[b], xyzv)
    iota = lax.iota(jnp.int32, L)
    inf = jnp.broadcast_to(jnp.float32(jnp.inf), (L,))

    def build(c, carry):
        s = c * L
        m = iota + _splat(s)
        xs = plsc.load_gather(xyzv, [m, _splat(0)])
        ys = plsc.load_gather(xyzv, [m, _splat(1)])
        zs = plsc.load_gather(xyzv, [m, _splat(2)])
        x2v[pl.ds(s, L)] = xs * xs + ys * ys + zs * zs
        xbv[pl.ds(s, L)] = _bf16_round(xs)
        ybv[pl.ds(s, L)] = _bf16_round(ys)
        zbv[pl.ds(s, L)] = _bf16_round(zs)
        return carry

    lax.fori_loop(0, N // L, build, 0)

    GQ = 4

    def per_group(g, carry):
        n0 = base + g * GQ
        qs = []
        for t in range(GQ):
            nsp = _splat(n0 + t)
            qs.append((plsc.load_gather(x2v, [nsp]),
                       plsc.load_gather(xbv, [nsp]),
                       plsc.load_gather(ybv, [nsp]),
                       plsc.load_gather(zbv, [nsp])))

        def chunk(j, st):
            s = j * L
            xb = xbv[pl.ds(s, L)]
            yb = ybv[pl.ds(s, L)]
            zb = zbv[pl.ds(s, L)]
            c2 = x2v[pl.ds(s, L)]
            d2s, hits = [], []
            anyhit = None
            for t in range(GQ):
                q2, qx, qy, qz = qs[t]
                dot = xb * qx + yb * qy + zb * qz
                d2 = jnp.maximum((q2 + c2) - 2.0 * dot, 0.0)
                d2s.append(d2)
                m = d2 < st[t][2]
                hits.append(m)
                anyhit = m if anyhit is None else (anyhit | m)

            def do_merges(st):
                out = []
                for t in range(GQ):
                    def merge_t(stt, t=t):
                        T, TI, _ = stt
                        ck, ci = plsc.sort_key_val(d2s[t], iota + _splat(s))
                        rk = lax.rev(ck, (0,))
                        ri = lax.rev(ci, (0,))
                        m = jnp.where(T <= rk, jnp.int32(-1), jnp.int32(0))
                        tb = plsc.bitcast(T, jnp.int32)
                        rb = plsc.bitcast(rk, jnp.int32)
                        nk = plsc.bitcast((tb & m) | (rb & ~m), jnp.float32)
                        ni = (TI & m) | (ri & ~m)
                        T2, TI2 = plsc.sort_key_val(nk, ni)
                        return T2, TI2, jnp.broadcast_to(jnp.max(T2), (L,))

                    out.append(lax.cond(jnp.any(hits[t]), merge_t,
                                        lambda s_: s_, st[t]))
                return tuple(out)

            return lax.cond(jnp.any(anyhit), do_merges, lambda s_: s_, st)

        init = tuple((inf, iota, inf) for _ in range(GQ))
        st = lax.fori_loop(0, N // L, chunk, init)
        for t in range(GQ):
            T, TI, _ = st[t]
            oidx[g * GQ + t] = TI
            od2[g * GQ + t] = T
        return carry

    lax.fori_loop(0, QPT // GQ, per_group, 0)
    pltpu.sync_copy(oidx, idx_hbm.at[b, pl.ds(base, QPT)])
    pltpu.sync_copy(od2, d2_hbm.at[b, pl.ds(base, QPT)])


@functools.partial(
    pl.kernel,
    mesh=_mesh,
    compiler_params=pltpu.CompilerParams(needs_layout_passes=False, use_tc_tiling_on_sc=False),
    out_type=jax.ShapeDtypeStruct((B, N, K, CH), jnp.float32),
    scratch_types=[
        pltpu.VMEM((N, 3), jnp.float32),
        pltpu.VMEM((N,), jnp.float32),
        pltpu.VMEM((N,), jnp.float32),
        pltpu.VMEM((N,), jnp.float32),
        pltpu.VMEM((N,), jnp.float32),
        pltpu.VMEM((N, K), jnp.int32),
        pltpu.VMEM((N, K), jnp.float32),
        pltpu.VMEM((N, 8), jnp.float32),
        pltpu.VMEM((QB, K, CH), jnp.float32),
    ],
)
def _feat_kernel(xyz_hbm, idx_hbm, d2_hbm, feat_hbm,
                 xyzv, x2v, xbv, ybv, zbv, idxv, d2v, intrav, outv):
    wid = _worker_id()
    b = wid // (N // QPT)
    base = (wid % (N // QPT)) * QPT
    pltpu.sync_copy(xyz_hbm.at[b], xyzv)
    pltpu.sync_copy(idx_hbm.at[b], idxv)
    pltpu.sync_copy(d2_hbm.at[b], d2v)
    iota = lax.iota(jnp.int32, L)

    def build(c, carry):
        s = c * L
        m = iota + _splat(s)
        xs = plsc.load_gather(xyzv, [m, _splat(0)])
        ys = plsc.load_gather(xyzv, [m, _splat(1)])
        zs = plsc.load_gather(xyzv, [m, _splat(2)])
        x2v[pl.ds(s, L)] = xs * xs + ys * ys + zs * zs
        xbv[pl.ds(s, L)] = _bf16_round(xs)
        ybv[pl.ds(s, L)] = _bf16_round(ys)
        zbv[pl.ds(s, L)] = _bf16_round(zs)
        return carry

    lax.fori_loop(0, N // L, build, 0)

    def gxyz(idx):
        return (plsc.load_gather(xbv, [idx]),
                plsc.load_gather(ybv, [idx]),
                plsc.load_gather(zbv, [idx]),
                plsc.load_gather(x2v, [idx]))

    def d2_of(p, q):
        dot = p[0] * q[0] + p[1] * q[1] + p[2] * q[2]
        return jnp.maximum((p[3] + q[3]) - 2.0 * dot, 0.0)

    def intra_chunk(c, carry):
        m = iota + _splat(c * L)
        anc = [plsc.load_gather(idxv, [m, _splat(i)]) for i in (1, 2, 3)]
        p = [gxyz(a) for a in anc]
        for ci in range(3):
            g = plsc.load_gather(d2v, [m, _splat(ci + 1)])
            plsc.store_scatter(intrav, [m, _splat(ci)], g)
        pairs = ((0, 1), (0, 2), (1, 2))
        for ci, (i, j) in enumerate(pairs):
            plsc.store_scatter(intrav, [m, _splat(3 + ci)], d2_of(p[i], p[j]))
        return carry

    lax.fori_loop(0, N // L, intra_chunk, 0)

    def per_block(g, carry):
        def per_query(q, carry2):
            n = base + g * QB + q
            nsp = _splat(n)
            nbr = idxv[n]
            qs = _splat(q)
            own = []
            for i in range(A):
                ai = plsc.load_gather(idxv, [nsp, _splat(i)])
                own.append(gxyz(ai))
            na = []
            for j in range(A):
                bj = plsc.load_gather(idxv, [nbr, _splat(j)])
                na.append(gxyz(bj))
            for ci in range(6):
                v = plsc.load_gather(intrav, [nsp, _splat(ci)])
                plsc.store_scatter(outv, [qs, iota, _splat(ci)], v)
            for ci in range(6):
                v = plsc.load_gather(intrav, [nbr, _splat(ci)])
                plsc.store_scatter(outv, [qs, iota, _splat(6 + ci)], v)
            for i in range(A):
                for j in range(A):
                    v = d2_of(own[i], na[j])
                    plsc.store_scatter(
                        outv, [qs, iota, _splat(12 + i * A + j)], v)
            return carry2

        lax.fori_loop(0, QB, per_query, 0)
        pltpu.sync_copy(outv, feat_hbm.at[b, pl.ds(base + g * QB, QB)])
        return carry

    lax.fori_loop(0, QPT // QB, per_block, 0)


_SQRT_ROWS = (B * N * K * CH) // 128
_SQRT_BLK = 2048


def _sqrt_body(x_ref, o_ref):
    o_ref[...] = jnp.sqrt(x_ref[...])


_sqrt_call = pl.pallas_call(
    _sqrt_body,
    out_shape=jax.ShapeDtypeStruct((_SQRT_ROWS, 128), jnp.float32),
    grid=(_SQRT_ROWS // _SQRT_BLK,),
    in_specs=[pl.BlockSpec((_SQRT_BLK, 128), lambda i: (i, 0))],
    out_specs=pl.BlockSpec((_SQRT_BLK, 128), lambda i: (i, 0)),
)


def kernel(xyz):
    idx, d2 = _topk_kernel(xyz)
    feat_d2 = _feat_kernel(xyz, idx, d2)
    feat = _sqrt_call(feat_d2.reshape(_SQRT_ROWS, 128)).reshape(B, N, K, CH)
    return feat, idx


# speedup vs baseline: 1.0787x; 1.0102x over previous
import functools

import jax
import jax.numpy as jnp
from jax import lax
from jax.experimental import pallas as pl
from jax.experimental.pallas import tpu as pltpu
from jax.experimental.pallas import tpu_sc as plsc

B = 8
N = 2048
K = 16
A = 4
CH = 28
L = 16
NWORKERS = 32
QPT = (B * N) // NWORKERS
QB = 16

_mesh = plsc.VectorSubcoreMesh(core_axis_name="c", subcore_axis_name="s")


def _worker_id():
    return lax.axis_index("s") * 2 + lax.axis_index("c")


def _splat(x, dtype=jnp.int32):
    return jnp.broadcast_to(jnp.asarray(x, dtype), (L,))


def _bf16_round(v):
    b = plsc.bitcast(v, jnp.int32)
    lsb = (b >> 16) & 1
    r = (b + 0x7FFF + lsb) & jnp.int32(-65536)
    return plsc.bitcast(r, jnp.float32)


@functools.partial(
    pl.kernel,
    mesh=_mesh,
    compiler_params=pltpu.CompilerParams(needs_layout_passes=False, use_tc_tiling_on_sc=False),
    out_type=[
        jax.ShapeDtypeStruct((B, N, K), jnp.int32),
        jax.ShapeDtypeStruct((B, N, K), jnp.float32),
    ],
    scratch_types=[
        pltpu.VMEM((N, 3), jnp.float32),
        pltpu.VMEM((N,), jnp.float32),
        pltpu.VMEM((N,), jnp.float32),
        pltpu.VMEM((N,), jnp.float32),
        pltpu.VMEM((N,), jnp.float32),
        pltpu.VMEM((QPT, K), jnp.int32),
        pltpu.VMEM((QPT, K), jnp.float32),
    ],
)
def _topk_kernel(xyz_hbm, idx_hbm, d2_hbm,
                 xyzv, x2v, xbv, ybv, zbv, oidx, od2):
    wid = _worker_id()
    b = wid // (N // QPT)
    base = (wid % (N // QPT)) * QPT
    pltpu.sync_copy(xyz_hbm.at[b], xyzv)
    iota = lax.iota(jnp.int32, L)
    inf = jnp.broadcast_to(jnp.float32(jnp.inf), (L,))

    def build(c, carry):
        s = c * L
        m = iota + _splat(s)
        xs = plsc.load_gather(xyzv, [m, _splat(0)])
        ys = plsc.load_gather(xyzv, [m, _splat(1)])
        zs = plsc.load_gather(xyzv, [m, _splat(2)])
        x2v[pl.ds(s, L)] = xs * xs + ys * ys + zs * zs
        xbv[pl.ds(s, L)] = _bf16_round(xs)
        ybv[pl.ds(s, L)] = _bf16_round(ys)
        zbv[pl.ds(s, L)] = _bf16_round(zs)
        return carry

    lax.fori_loop(0, N // L, build, 0)

    GQ = 4

    def per_group(g, carry):
        n0 = base + g * GQ
        qs = []
        for t in range(GQ):
            nsp = _splat(n0 + t)
            qs.append((plsc.load_gather(x2v, [nsp]),
                       plsc.load_gather(xbv, [nsp]),
                       plsc.load_gather(ybv, [nsp]),
                       plsc.load_gather(zbv, [nsp])))

        def chunk(j, st):
            s = j * L
            xb = xbv[pl.ds(s, L)]
            yb = ybv[pl.ds(s, L)]
            zb = zbv[pl.ds(s, L)]
            c2 = x2v[pl.ds(s, L)]
            d2s, hits = [], []
            anyhit = None
            for t in range(GQ):
                q2, qx, qy, qz = qs[t]
                dot = xb * qx + yb * qy + zb * qz
                d2 = jnp.maximum((q2 + c2) - 2.0 * dot, 0.0)
                d2s.append(d2)
                m = d2 < st[t][2]
                hits.append(m)
                anyhit = m if anyhit is None else (anyhit | m)

            def do_merges(st):
                out = []
                for t in range(GQ):
                    def merge_t(stt, t=t):
                        T, TI, _ = stt
                        ck, ci = plsc.sort_key_val(d2s[t], iota + _splat(s))
                        rk = lax.rev(ck, (0,))
                        ri = lax.rev(ci, (0,))
                        m = jnp.where(T <= rk, jnp.int32(-1), jnp.int32(0))
                        tb = plsc.bitcast(T, jnp.int32)
                        rb = plsc.bitcast(rk, jnp.int32)
                        nk = plsc.bitcast((tb & m) | (rb & ~m), jnp.float32)
                        ni = (TI & m) | (ri & ~m)
                        T2, TI2 = plsc.sort_key_val(nk, ni)
                        return T2, TI2, jnp.broadcast_to(jnp.max(T2), (L,))

                    out.append(lax.cond(jnp.any(hits[t]), merge_t,
                                        lambda s_: s_, st[t]))
                return tuple(out)

            return lax.cond(jnp.any(anyhit), do_merges, lambda s_: s_, st)

        init = tuple((inf, iota, inf) for _ in range(GQ))
        st = lax.fori_loop(0, N // L, chunk, init, unroll=2)
        for t in range(GQ):
            T, TI, _ = st[t]
            oidx[g * GQ + t] = TI
            od2[g * GQ + t] = T
        return carry

    lax.fori_loop(0, QPT // GQ, per_group, 0)
    pltpu.sync_copy(oidx, idx_hbm.at[b, pl.ds(base, QPT)])
    pltpu.sync_copy(od2, d2_hbm.at[b, pl.ds(base, QPT)])


@functools.partial(
    pl.kernel,
    mesh=_mesh,
    compiler_params=pltpu.CompilerParams(needs_layout_passes=False, use_tc_tiling_on_sc=False),
    out_type=jax.ShapeDtypeStruct((B, N, K, CH), jnp.float32),
    scratch_types=[
        pltpu.VMEM((N, 3), jnp.float32),
        pltpu.VMEM((N,), jnp.float32),
        pltpu.VMEM((N,), jnp.float32),
        pltpu.VMEM((N,), jnp.float32),
        pltpu.VMEM((N,), jnp.float32),
        pltpu.VMEM((N, K), jnp.int32),
        pltpu.VMEM((N, K), jnp.float32),
        pltpu.VMEM((N, 8), jnp.float32),
        pltpu.VMEM((QB, K, CH), jnp.float32),
    ],
)
def _feat_kernel(xyz_hbm, idx_hbm, d2_hbm, feat_hbm,
                 xyzv, x2v, xbv, ybv, zbv, idxv, d2v, intrav, outv):
    wid = _worker_id()
    b = wid // (N // QPT)
    base = (wid % (N // QPT)) * QPT
    pltpu.sync_copy(xyz_hbm.at[b], xyzv)
    pltpu.sync_copy(idx_hbm.at[b], idxv)
    pltpu.sync_copy(d2_hbm.at[b], d2v)
    iota = lax.iota(jnp.int32, L)

    def build(c, carry):
        s = c * L
        m = iota + _splat(s)
        xs = plsc.load_gather(xyzv, [m, _splat(0)])
        ys = plsc.load_gather(xyzv, [m, _splat(1)])
        zs = plsc.load_gather(xyzv, [m, _splat(2)])
        x2v[pl.ds(s, L)] = xs * xs + ys * ys + zs * zs
        xbv[pl.ds(s, L)] = _bf16_round(xs)
        ybv[pl.ds(s, L)] = _bf16_round(ys)
        zbv[pl.ds(s, L)] = _bf16_round(zs)
        return carry

    lax.fori_loop(0, N // L, build, 0)

    def gxyz(idx):
        return (plsc.load_gather(xbv, [idx]),
                plsc.load_gather(ybv, [idx]),
                plsc.load_gather(zbv, [idx]),
                plsc.load_gather(x2v, [idx]))

    def d2_of(p, q):
        dot = p[0] * q[0] + p[1] * q[1] + p[2] * q[2]
        return jnp.maximum((p[3] + q[3]) - 2.0 * dot, 0.0)

    def intra_chunk(c, carry):
        m = iota + _splat(c * L)
        anc = [plsc.load_gather(idxv, [m, _splat(i)]) for i in (1, 2, 3)]
        p = [gxyz(a) for a in anc]
        for ci in range(3):
            g = plsc.load_gather(d2v, [m, _splat(ci + 1)])
            plsc.store_scatter(intrav, [m, _splat(ci)], g)
        pairs = ((0, 1), (0, 2), (1, 2))
        for ci, (i, j) in enumerate(pairs):
            plsc.store_scatter(intrav, [m, _splat(3 + ci)], d2_of(p[i], p[j]))
        return carry

    lax.fori_loop(0, N // L, intra_chunk, 0)

    def per_block(g, carry):
        def per_query(q, carry2):
            n = base + g * QB + q
            nsp = _splat(n)
            nbr = idxv[n]
            qs = _splat(q)
            own = []
            for i in range(A):
                ai = plsc.load_gather(idxv, [nsp, _splat(i)])
                own.append(gxyz(ai))
            na = []
            for j in range(A):
                bj = plsc.load_gather(idxv, [nbr, _splat(j)])
                na.append(gxyz(bj))
            for ci in range(6):
                v = plsc.load_gather(intrav, [nsp, _splat(ci)])
                plsc.store_scatter(outv, [qs, iota, _splat(ci)], v)
            for ci in range(6):
                v = plsc.load_gather(intrav, [nbr, _splat(ci)])
                plsc.store_scatter(outv, [qs, iota, _splat(6 + ci)], v)
            for i in range(A):
                for j in range(A):
                    v = d2_of(own[i], na[j])
                    plsc.store_scatter(
                        outv, [qs, iota, _splat(12 + i * A + j)], v)
            return carry2

        lax.fori_loop(0, QB, per_query, 0)
        pltpu.sync_copy(outv, feat_hbm.at[b, pl.ds(base + g * QB, QB)])
        return carry

    lax.fori_loop(0, QPT // QB, per_block, 0)


_SQRT_ROWS = (B * N * K * CH) // 128
_SQRT_BLK = 2048


def _sqrt_body(x_ref, o_ref):
    o_ref[...] = jnp.sqrt(x_ref[...])


_sqrt_call = pl.pallas_call(
    _sqrt_body,
    out_shape=jax.ShapeDtypeStruct((_SQRT_ROWS, 128), jnp.float32),
    grid=(_SQRT_ROWS // _SQRT_BLK,),
    in_specs=[pl.BlockSpec((_SQRT_BLK, 128), lambda i: (i, 0))],
    out_specs=pl.BlockSpec((_SQRT_BLK, 128), lambda i: (i, 0)),
)


def kernel(xyz):
    idx, d2 = _topk_kernel(xyz)
    feat_d2 = _feat_kernel(xyz, idx, d2)
    feat = _sqrt_call(feat_d2.reshape(_SQRT_ROWS, 128)).reshape(B, N, K, CH)
    return feat, idx
